# Initial kernel scaffold; baseline (speedup 1.0000x reference)
import jax
import jax.numpy as jnp
from jax.experimental import pallas as pl


def kernel(x, edge_index, batch, W, att_src, att_dst, bias, lin1_W, lin1_b):
    raise NotImplementedError("write your pallas kernel here")


# speedup vs baseline: 41.7613x; 41.7613x over previous
import functools

import jax
import jax.numpy as jnp
from jax import lax
from jax.experimental import pallas as pl
from jax.experimental.pallas import tpu as pltpu
from jax.experimental.pallas import tpu_sc as plsc

N = 10240
E = 327680
IN = 128
HID = 64
G = 80
NPG = 128

NC = 2
NS = 16
NW = NC * NS
EPW = E // NW
BE = 128
NB = EPW // BE
RPT = N // NS


def _k1_body(x_ref, w_ref, as_ref, ad_ref, h_ref, av_ref, bv_ref, m_ref, smax):
    i = pl.program_id(0)
    h = jnp.dot(x_ref[...], w_ref[...], preferred_element_type=jnp.float32)
    h_ref[...] = h
    a_s = jnp.sum(h * as_ref[...], axis=1, keepdims=True)
    a_d = jnp.sum(h * ad_ref[...], axis=1, keepdims=True)
    av_ref[...] = a_s
    bv_ref[...] = a_d
    bs = jnp.max(a_s)
    bd = jnp.max(a_d)

    @pl.when(i == 0)
    def _():
        smax[0] = bs
        smax[1] = bd

    @pl.when(i > 0)
    def _():
        smax[0] = jnp.maximum(smax[0], bs)
        smax[1] = jnp.maximum(smax[1], bd)

    @pl.when(i == pl.num_programs(0) - 1)
    def _():
        mm = smax[0] + smax[1]
        mm = jnp.where(mm >= 0.0, mm, 0.2 * mm)
        m_ref[...] = jnp.full((16,), mm, jnp.float32)


def _k1(x, W, att_src, att_dst):
    blk = 1024
    grid = N // blk
    return pl.pallas_call(
        _k1_body,
        grid=(grid,),
        in_specs=[
            pl.BlockSpec((blk, IN), lambda i: (i, 0)),
            pl.BlockSpec((IN, HID), lambda i: (0, 0)),
            pl.BlockSpec((1, HID), lambda i: (0, 0)),
            pl.BlockSpec((1, HID), lambda i: (0, 0)),
        ],
        out_specs=[
            pl.BlockSpec((blk, HID), lambda i: (i, 0)),
            pl.BlockSpec((blk, 1), lambda i: (i, 0)),
            pl.BlockSpec((blk, 1), lambda i: (i, 0)),
            pl.BlockSpec((16,), lambda i: (0,)),
        ],
        out_shape=[
            jax.ShapeDtypeStruct((N, HID), jnp.float32),
            jax.ShapeDtypeStruct((N, 1), jnp.float32),
            jax.ShapeDtypeStruct((N, 1), jnp.float32),
            jax.ShapeDtypeStruct((16,), jnp.float32),
        ],
        scratch_shapes=[pltpu.SMEM((2,), jnp.float32)],
    )(x, W, att_src, att_dst)


def _k2_body(src_hbm, dst_hbm, asrc_hbm, adst_hbm, m_hbm, h_hbm,
             num_out, den_out,
             src_buf, dst_buf, asrc_t, adst_t, m_buf, den_t, w_buf,
             rows, zbuf, num_sh, sem):
    c = lax.axis_index("c")
    s = lax.axis_index("s")
    wid = c * NS + s

    pltpu.sync_copy(src_hbm.at[wid], src_buf)
    pltpu.sync_copy(dst_hbm.at[wid], dst_buf)
    pltpu.sync_copy(asrc_hbm, asrc_t)
    pltpu.sync_copy(adst_hbm, adst_t)
    pltpu.sync_copy(m_hbm, m_buf)
    m_v = m_buf[...]

    zero16 = jnp.zeros((16,), jnp.float32)

    def _z(j, _):
        den_t[pl.ds(j * 16, 16)] = zero16
        return 0
    lax.fori_loop(0, N // 16, _z, 0)

    def _zz(b, _):
        for k in range(HID // 16):
            zbuf[b, pl.ds(k * 16, 16)] = zero16
        return 0
    lax.fori_loop(0, BE, _zz, 0)

    for j in range(RPT // BE):
        pltpu.sync_copy(zbuf, num_sh.at[pl.ds(s * RPT + j * BE, BE)])
    plsc.subcore_barrier()

    def _blk(b, _):
        gat = pltpu.make_async_copy(h_hbm.at[src_buf.at[b]], rows, sem)
        gat.start()
        for g in range(BE // 16):
            sl = pl.ds(g * 16, 16)
            si = src_buf[b, sl]
            di = dst_buf[b, sl]
            al = plsc.load_gather(asrc_t, [si]) + plsc.load_gather(adst_t, [di])
            al = jnp.where(al >= 0.0, al, 0.2 * al)
            w = jnp.exp(al - m_v)
            w_buf[sl] = w
            plsc.addupdate_scatter(den_t, [di], w)
        gat.wait()

        def _esc(g, _):
            wv = w_buf[pl.ds(g * 16, 16)]
            for i in range(16):
                ws = wv[i]
                e = g * 16 + i
                for k in range(HID // 16):
                    sl = pl.ds(k * 16, 16)
                    rows[e, sl] = rows[e, sl] * ws
            return 0
        lax.fori_loop(0, BE // 16, _esc, 0)

        pltpu.sync_copy(rows, num_sh.at[dst_buf.at[b]], add=True)
        return 0
    lax.fori_loop(0, NB, _blk, 0)

    pltpu.sync_copy(den_t, den_out.at[wid])
    plsc.subcore_barrier()

    for j in range(RPT // BE):
        r0 = s * RPT + j * BE
        pltpu.sync_copy(num_sh.at[pl.ds(r0, BE)], rows)
        pltpu.sync_copy(rows, num_out.at[c].at[pl.ds(r0, BE)])


def _k2(src3, dst3, asrc, adst, m8, h):
    mesh = plsc.VectorSubcoreMesh(core_axis_name="c", subcore_axis_name="s")
    f = pl.kernel(
        _k2_body,
        out_type=[
            jax.ShapeDtypeStruct((NC, N, HID), jnp.float32),
            jax.ShapeDtypeStruct((NW, N), jnp.float32),
        ],
        mesh=mesh,
        compiler_params=pltpu.CompilerParams(
            needs_layout_passes=False, use_tc_tiling_on_sc=False),
        scratch_types=[
            pltpu.VMEM((NB, BE), jnp.int32),
            pltpu.VMEM((NB, BE), jnp.int32),
            pltpu.VMEM((N,), jnp.float32),
            pltpu.VMEM((N,), jnp.float32),
            pltpu.VMEM((16,), jnp.float32),
            pltpu.VMEM((N,), jnp.float32),
            pltpu.VMEM((BE,), jnp.float32),
            pltpu.VMEM((BE, HID), jnp.float32),
            pltpu.VMEM((BE, HID), jnp.float32),
            pltpu.VMEM_SHARED((N, HID), jnp.float32),
            pltpu.SemaphoreType.DMA,
        ],
    )
    return f(src3, dst3, asrc, adst, m8, h)


def _k3_body(num_ref, denT_ref, bias_ref, w2T_ref, b2_ref, out_ref):
    n = num_ref[0] + num_ref[1]
    d = jnp.sum(denT_ref[...], axis=1, keepdims=True)
    vals = n / (d + 1e-16) + bias_ref[...]
    vals = jnp.maximum(vals, 0.0)
    z = jnp.dot(vals, w2T_ref[...], preferred_element_type=jnp.float32)
    ri = lax.broadcasted_iota(jnp.int32, (N, NPG), 0)
    ci = lax.broadcasted_iota(jnp.int32, (N, NPG), 1)
    rowsum = jnp.sum(jnp.where((ri % NPG) == ci, z, 0.0), axis=1,
                     keepdims=True)
    gi = lax.broadcasted_iota(jnp.int32, (G, N), 0)
    rg = lax.broadcasted_iota(jnp.int32, (G, N), 1) // NPG
    sel = (gi == rg).astype(jnp.float32)
    y = jnp.dot(sel, rowsum, preferred_element_type=jnp.float32) + b2_ref[0, 0]
    out_ref[...] = 1.0 / (1.0 + jnp.exp(-y))


def _k3(num, denT, bias, w2T, b2):
    return pl.pallas_call(
        _k3_body,
        out_shape=jax.ShapeDtypeStruct((G, 1), jnp.float32),
    )(num, denT, bias, w2T, b2)


@jax.jit
def kernel(x, edge_index, batch, W, att_src, att_dst, bias, lin1_W, lin1_b):
    src3 = edge_index[0].reshape(NW, NB, BE)
    dst3 = edge_index[1].reshape(NW, NB, BE)
    h, a_s, a_d, m8 = _k1(x, W, att_src.reshape(1, HID), att_dst.reshape(1, HID))
    num, den = _k2(src3, dst3, a_s.reshape(N), a_d.reshape(N), m8, h)
    w2T = lin1_W.reshape(NPG, HID).T
    return _k3(num, den.T, bias.reshape(1, HID), w2T, lin1_b.reshape(1, 1))


# speedup vs baseline: 100.5440x; 2.4076x over previous
import functools

import jax
import jax.numpy as jnp
from jax import lax
from jax.experimental import pallas as pl
from jax.experimental.pallas import tpu as pltpu
from jax.experimental.pallas import tpu_sc as plsc

N = 10240
E = 327680
IN = 128
HID = 64
G = 80
NPG = 128

NC = 2
NS = 16
NW = NC * NS
EPW = E // NW
BE = 128
NB = EPW // BE
RPT = N // NS


def _k1_body(x_ref, w_ref, as_ref, ad_ref, h_ref, av_ref, bv_ref, m_ref, smax):
    i = pl.program_id(0)
    h = jnp.dot(x_ref[...], w_ref[...], preferred_element_type=jnp.float32)
    h_ref[...] = h
    a_s = jnp.sum(h * as_ref[...], axis=1, keepdims=True)
    a_d = jnp.sum(h * ad_ref[...], axis=1, keepdims=True)
    av_ref[...] = a_s
    bv_ref[...] = a_d
    bs = jnp.max(a_s)
    bd = jnp.max(a_d)

    @pl.when(i == 0)
    def _():
        smax[0] = bs
        smax[1] = bd

    @pl.when(i > 0)
    def _():
        smax[0] = jnp.maximum(smax[0], bs)
        smax[1] = jnp.maximum(smax[1], bd)

    @pl.when(i == pl.num_programs(0) - 1)
    def _():
        mm = smax[0] + smax[1]
        mm = jnp.where(mm >= 0.0, mm, 0.2 * mm)
        m_ref[...] = jnp.full((16,), mm, jnp.float32)


def _k1(x, W, att_src, att_dst):
    blk = 1024
    grid = N // blk
    return pl.pallas_call(
        _k1_body,
        grid=(grid,),
        in_specs=[
            pl.BlockSpec((blk, IN), lambda i: (i, 0)),
            pl.BlockSpec((IN, HID), lambda i: (0, 0)),
            pl.BlockSpec((1, HID), lambda i: (0, 0)),
            pl.BlockSpec((1, HID), lambda i: (0, 0)),
        ],
        out_specs=[
            pl.BlockSpec((blk, HID), lambda i: (i, 0)),
            pl.BlockSpec((blk, 1), lambda i: (i, 0)),
            pl.BlockSpec((blk, 1), lambda i: (i, 0)),
            pl.BlockSpec((16,), lambda i: (0,)),
        ],
        out_shape=[
            jax.ShapeDtypeStruct((N, HID), jnp.float32),
            jax.ShapeDtypeStruct((N, 1), jnp.float32),
            jax.ShapeDtypeStruct((N, 1), jnp.float32),
            jax.ShapeDtypeStruct((16,), jnp.float32),
        ],
        scratch_shapes=[pltpu.SMEM((2,), jnp.float32)],
    )(x, W, att_src, att_dst)


NBUF = 2


def _k2_body(src_hbm, dst_hbm, asrc_hbm, adst_hbm, m_hbm, h_hbm,
             num_out, den_out,
             src_buf, dst_buf, asrc_t, adst_t, m_buf, den_t, w_buf,
             rows, srows, num_sh, *sems):
    gsem = sems[:NBUF]
    ssem = sems[NBUF:]
    c = lax.axis_index("c")
    s = lax.axis_index("s")
    wid = c * NS + s

    pltpu.sync_copy(src_hbm.at[wid], src_buf)
    pltpu.sync_copy(dst_hbm.at[wid], dst_buf)
    pltpu.sync_copy(asrc_hbm, asrc_t)
    pltpu.sync_copy(adst_hbm, adst_t)
    pltpu.sync_copy(m_hbm, m_buf)
    m_v = m_buf[...]

    zero16 = jnp.zeros((16,), jnp.float32)

    def _z(j, _):
        den_t[pl.ds(j * 16, 16)] = zero16
        return 0
    lax.fori_loop(0, N // 16, _z, 0)

    def _zz(b, _):
        for k in range(HID // 16):
            rows[0, b, pl.ds(k * 16, 16)] = zero16
        return 0
    lax.fori_loop(0, BE, _zz, 0)
    for j in range(RPT // BE):
        pltpu.sync_copy(rows.at[0], num_sh.at[pl.ds(s * RPT + j * BE, BE)])
    plsc.subcore_barrier()

    for j in range(NBUF):
        pltpu.make_async_copy(h_hbm.at[src_buf.at[j]], rows.at[j],
                              gsem[j]).start()

    def _iter(i, _):
        for j in range(NBUF):
            b = i * NBUF + j
            for g in range(BE // 16):
                sl = pl.ds(g * 16, 16)
                si = src_buf[b, sl]
                di = dst_buf[b, sl]
                al = (plsc.load_gather(asrc_t, [si])
                      + plsc.load_gather(adst_t, [di]))
                al = jnp.where(al >= 0.0, al, 0.2 * al)
                w = jnp.exp(al - m_v)
                w_buf[sl] = w
                plsc.addupdate_scatter(den_t, [di], w)

            pltpu.make_async_copy(h_hbm.at[src_buf.at[b]], rows.at[j],
                                  gsem[j]).wait()

            @pl.when(i > 0)
            def _():
                pltpu.make_async_copy(srows.at[j], num_sh.at[pl.ds(0, BE)],
                                      ssem[j]).wait()

            def _esc(g, _):
                wv = w_buf[pl.ds(g * 16, 16)]
                for e16 in range(16):
                    ws = wv[e16]
                    e = g * 16 + e16
                    for k in range(HID // 16):
                        sl = pl.ds(k * 16, 16)
                        srows[j, e, sl] = rows[j, e, sl] * ws
                return 0
            lax.fori_loop(0, BE // 16, _esc, 0)

            @pl.when(b + NBUF < NB)
            def _():
                pltpu.make_async_copy(h_hbm.at[src_buf.at[b + NBUF]],
                                      rows.at[j], gsem[j]).start()

            pltpu.async_copy(srows.at[j], num_sh.at[dst_buf.at[b]], ssem[j],
                             add=True)
        return 0
    lax.fori_loop(0, NB // NBUF, _iter, 0)

    for j in range(NBUF):
        pltpu.make_async_copy(srows.at[j], num_sh.at[pl.ds(0, BE)],
                              ssem[j]).wait()

    pltpu.sync_copy(den_t, den_out.at[wid])
    plsc.subcore_barrier()

    for j in range(RPT // BE):
        r0 = s * RPT + j * BE
        pltpu.sync_copy(num_sh.at[pl.ds(r0, BE)], rows.at[0])
        pltpu.sync_copy(rows.at[0], num_out.at[c].at[pl.ds(r0, BE)])


def _k2(src3, dst3, asrc, adst, m8, h):
    mesh = plsc.VectorSubcoreMesh(core_axis_name="c", subcore_axis_name="s")
    f = pl.kernel(
        _k2_body,
        out_type=[
            jax.ShapeDtypeStruct((NC, N, HID), jnp.float32),
            jax.ShapeDtypeStruct((NW, N), jnp.float32),
        ],
        mesh=mesh,
        compiler_params=pltpu.CompilerParams(
            needs_layout_passes=False, use_tc_tiling_on_sc=False),
        scratch_types=[
            pltpu.VMEM((NB, BE), jnp.int32),
            pltpu.VMEM((NB, BE), jnp.int32),
            pltpu.VMEM((N,), jnp.float32),
            pltpu.VMEM((N,), jnp.float32),
            pltpu.VMEM((16,), jnp.float32),
            pltpu.VMEM((N,), jnp.float32),
            pltpu.VMEM((BE,), jnp.float32),
            pltpu.VMEM((NBUF, BE, HID), jnp.float32),
            pltpu.VMEM((NBUF, BE, HID), jnp.float32),
            pltpu.VMEM_SHARED((N, HID), jnp.float32),
        ] + [pltpu.SemaphoreType.DMA] * (2 * NBUF),
    )
    return f(src3, dst3, asrc, adst, m8, h)


def _k3_body(num_ref, denT_ref, bias_ref, w2T_ref, b2_ref, out_ref):
    n = num_ref[0] + num_ref[1]
    d = jnp.sum(denT_ref[...], axis=1, keepdims=True)
    vals = n / (d + 1e-16) + bias_ref[...]
    vals = jnp.maximum(vals, 0.0)
    z = jnp.dot(vals, w2T_ref[...], preferred_element_type=jnp.float32)
    ri = lax.broadcasted_iota(jnp.int32, (N, NPG), 0)
    ci = lax.broadcasted_iota(jnp.int32, (N, NPG), 1)
    rowsum = jnp.sum(jnp.where((ri % NPG) == ci, z, 0.0), axis=1,
                     keepdims=True)
    gi = lax.broadcasted_iota(jnp.int32, (G, N), 0)
    rg = lax.broadcasted_iota(jnp.int32, (G, N), 1) // NPG
    sel = (gi == rg).astype(jnp.float32)
    y = jnp.dot(sel, rowsum, preferred_element_type=jnp.float32) + b2_ref[0, 0]
    out_ref[...] = 1.0 / (1.0 + jnp.exp(-y))


def _k3(num, denT, bias, w2T, b2):
    return pl.pallas_call(
        _k3_body,
        out_shape=jax.ShapeDtypeStruct((G, 1), jnp.float32),
    )(num, denT, bias, w2T, b2)


@jax.jit
def kernel(x, edge_index, batch, W, att_src, att_dst, bias, lin1_W, lin1_b):
    src3 = edge_index[0].reshape(NW, NB, BE)
    dst3 = edge_index[1].reshape(NW, NB, BE)
    h, a_s, a_d, m8 = _k1(x, W, att_src.reshape(1, HID), att_dst.reshape(1, HID))
    num, den = _k2(src3, dst3, a_s.reshape(N), a_d.reshape(N), m8, h)
    w2T = lin1_W.reshape(NPG, HID).T
    return _k3(num, den.T, bias.reshape(1, HID), w2T, lin1_b.reshape(1, 1))


# speedup vs baseline: 104.5203x; 1.0395x over previous
import functools

import jax
import jax.numpy as jnp
from jax import lax
from jax.experimental import pallas as pl
from jax.experimental.pallas import tpu as pltpu
from jax.experimental.pallas import tpu_sc as plsc

N = 10240
E = 327680
IN = 128
HID = 64
G = 80
NPG = 128

NC = 2
NS = 16
NW = NC * NS
EPW = E // NW
BE = 128
NB = EPW // BE
RPT = N // NS


def _k1_body(x_ref, w_ref, as_ref, ad_ref, h_ref, av_ref, bv_ref, m_ref, smax):
    i = pl.program_id(0)
    h = jnp.dot(x_ref[...], w_ref[...], preferred_element_type=jnp.float32)
    h_ref[...] = h
    a_s = jnp.sum(h * as_ref[...], axis=1, keepdims=True)
    a_d = jnp.sum(h * ad_ref[...], axis=1, keepdims=True)
    av_ref[...] = a_s
    bv_ref[...] = a_d
    bs = jnp.max(a_s)
    bd = jnp.max(a_d)

    @pl.when(i == 0)
    def _():
        smax[0] = bs
        smax[1] = bd

    @pl.when(i > 0)
    def _():
        smax[0] = jnp.maximum(smax[0], bs)
        smax[1] = jnp.maximum(smax[1], bd)

    @pl.when(i == pl.num_programs(0) - 1)
    def _():
        mm = smax[0] + smax[1]
        mm = jnp.where(mm >= 0.0, mm, 0.2 * mm)
        m_ref[...] = jnp.full((16,), mm, jnp.float32)


def _k1(x, W, att_src, att_dst):
    blk = 1024
    grid = N // blk
    return pl.pallas_call(
        _k1_body,
        grid=(grid,),
        in_specs=[
            pl.BlockSpec((blk, IN), lambda i: (i, 0)),
            pl.BlockSpec((IN, HID), lambda i: (0, 0)),
            pl.BlockSpec((1, HID), lambda i: (0, 0)),
            pl.BlockSpec((1, HID), lambda i: (0, 0)),
        ],
        out_specs=[
            pl.BlockSpec((blk, HID), lambda i: (i, 0)),
            pl.BlockSpec((blk, 1), lambda i: (i, 0)),
            pl.BlockSpec((blk, 1), lambda i: (i, 0)),
            pl.BlockSpec((16,), lambda i: (0,)),
        ],
        out_shape=[
            jax.ShapeDtypeStruct((N, HID), jnp.float32),
            jax.ShapeDtypeStruct((N, 1), jnp.float32),
            jax.ShapeDtypeStruct((N, 1), jnp.float32),
            jax.ShapeDtypeStruct((16,), jnp.float32),
        ],
        scratch_shapes=[pltpu.SMEM((2,), jnp.float32)],
    )(x, W, att_src, att_dst)


NBUF = 2


def _k2_body(ei_hbm, asrc_hbm, adst_hbm, m_hbm, h_hbm,
             num_out, den_out,
             src_buf, dst_buf, asrc_t, adst_t, m_buf, den_t, w_buf,
             rows, srows, num_sh, *sems):
    gsem = sems[:NBUF]
    ssem = sems[NBUF:]
    c = lax.axis_index("c")
    s = lax.axis_index("s")
    wid = c * NS + s

    pltpu.sync_copy(ei_hbm.at[0].at[wid], src_buf)
    pltpu.sync_copy(ei_hbm.at[1].at[wid], dst_buf)
    pltpu.sync_copy(asrc_hbm, asrc_t)
    pltpu.sync_copy(adst_hbm, adst_t)
    pltpu.sync_copy(m_hbm, m_buf)
    m_v = m_buf[...]

    zero16 = jnp.zeros((16,), jnp.float32)

    def _z(j, _):
        den_t[pl.ds(j * 16, 16)] = zero16
        return 0
    lax.fori_loop(0, N // 16, _z, 0)

    def _zz(b, _):
        for k in range(HID // 16):
            rows[0, b, pl.ds(k * 16, 16)] = zero16
        return 0
    lax.fori_loop(0, BE, _zz, 0)
    for j in range(RPT // BE):
        pltpu.sync_copy(rows.at[0], num_sh.at[pl.ds(s * RPT + j * BE, BE)])
    plsc.subcore_barrier()

    for j in range(NBUF):
        pltpu.make_async_copy(h_hbm.at[src_buf.at[j]], rows.at[j],
                              gsem[j]).start()

    def _iter(i, _):
        for j in range(NBUF):
            b = i * NBUF + j
            for g in range(BE // 16):
                sl = pl.ds(g * 16, 16)
                si = src_buf[b, sl]
                di = dst_buf[b, sl]
                al = (plsc.load_gather(asrc_t, [si])
                      + plsc.load_gather(adst_t, [di]))
                al = jnp.where(al >= 0.0, al, 0.2 * al)
                w = jnp.exp(al - m_v)
                w_buf[sl] = w
                plsc.addupdate_scatter(den_t, [di], w)

            pltpu.make_async_copy(h_hbm.at[src_buf.at[b]], rows.at[j],
                                  gsem[j]).wait()

            @pl.when(i > 0)
            def _():
                pltpu.make_async_copy(srows.at[j], num_sh.at[pl.ds(0, BE)],
                                      ssem[j]).wait()

            for g in range(BE // 16):
                wv = w_buf[pl.ds(g * 16, 16)]
                for e16 in range(16):
                    ws = wv[e16]
                    e = g * 16 + e16
                    for k in range(HID // 16):
                        sl = pl.ds(k * 16, 16)
                        srows[j, e, sl] = rows[j, e, sl] * ws

            @pl.when(b + NBUF < NB)
            def _():
                pltpu.make_async_copy(h_hbm.at[src_buf.at[b + NBUF]],
                                      rows.at[j], gsem[j]).start()

            pltpu.async_copy(srows.at[j], num_sh.at[dst_buf.at[b]], ssem[j],
                             add=True)
        return 0
    lax.fori_loop(0, NB // NBUF, _iter, 0)

    for j in range(NBUF):
        pltpu.make_async_copy(srows.at[j], num_sh.at[pl.ds(0, BE)],
                              ssem[j]).wait()

    pltpu.sync_copy(den_t, den_out.at[wid])
    plsc.subcore_barrier()

    for j in range(RPT // BE):
        r0 = s * RPT + j * BE
        pltpu.sync_copy(num_sh.at[pl.ds(r0, BE)], rows.at[0])
        pltpu.sync_copy(rows.at[0], num_out.at[c].at[pl.ds(r0, BE)])


def _k2(ei4, asrc, adst, m8, h):
    mesh = plsc.VectorSubcoreMesh(core_axis_name="c", subcore_axis_name="s")
    f = pl.kernel(
        _k2_body,
        out_type=[
            jax.ShapeDtypeStruct((NC, N, HID), jnp.float32),
            jax.ShapeDtypeStruct((NW, N), jnp.float32),
        ],
        mesh=mesh,
        compiler_params=pltpu.CompilerParams(
            needs_layout_passes=False, use_tc_tiling_on_sc=False),
        scratch_types=[
            pltpu.VMEM((NB, BE), jnp.int32),
            pltpu.VMEM((NB, BE), jnp.int32),
            pltpu.VMEM((N,), jnp.float32),
            pltpu.VMEM((N,), jnp.float32),
            pltpu.VMEM((16,), jnp.float32),
            pltpu.VMEM((N,), jnp.float32),
            pltpu.VMEM((BE,), jnp.float32),
            pltpu.VMEM((NBUF, BE, HID), jnp.float32),
            pltpu.VMEM((NBUF, BE, HID), jnp.float32),
            pltpu.VMEM_SHARED((N, HID), jnp.float32),
        ] + [pltpu.SemaphoreType.DMA] * (2 * NBUF),
    )
    return f(ei4, asrc, adst, m8, h)


def _k3_body(num_ref, den_ref, bias_ref, w2T_ref, b2_ref, out_ref):
    n = num_ref[0] + num_ref[1]
    dsum = jnp.sum(den_ref[...], axis=0, keepdims=True)
    d = jnp.transpose(dsum)
    vals = n / (d + 1e-16) + bias_ref[...]
    vals = jnp.maximum(vals, 0.0)
    z = jnp.dot(vals, w2T_ref[...], preferred_element_type=jnp.float32)
    ri = lax.broadcasted_iota(jnp.int32, (N, NPG), 0)
    ci = lax.broadcasted_iota(jnp.int32, (N, NPG), 1)
    rowsum = jnp.sum(jnp.where((ri % NPG) == ci, z, 0.0), axis=1,
                     keepdims=True)
    gi = lax.broadcasted_iota(jnp.int32, (G, N), 0)
    rg = lax.broadcasted_iota(jnp.int32, (G, N), 1) // NPG
    sel = (gi == rg).astype(jnp.float32)
    y = jnp.dot(sel, rowsum, preferred_element_type=jnp.float32) + b2_ref[0, 0]
    out_ref[...] = 1.0 / (1.0 + jnp.exp(-y))


def _k3(num, denT, bias, w2T, b2):
    return pl.pallas_call(
        _k3_body,
        out_shape=jax.ShapeDtypeStruct((G, 1), jnp.float32),
    )(num, denT, bias, w2T, b2)


@jax.jit
def kernel(x, edge_index, batch, W, att_src, att_dst, bias, lin1_W, lin1_b):
    ei4 = edge_index.reshape(2, NW, NB, BE)
    h, a_s, a_d, m8 = _k1(x, W, att_src.reshape(1, HID), att_dst.reshape(1, HID))
    num, den = _k2(ei4, a_s.reshape(N), a_d.reshape(N), m8, h)
    w2T = lin1_W.reshape(NPG, HID).T
    return _k3(num, den, bias.reshape(1, HID), w2T, lin1_b.reshape(1, 1))
